# Initial kernel scaffold; baseline (speedup 1.0000x reference)
import jax
import jax.numpy as jnp
from jax.experimental import pallas as pl


def kernel(x, v1, g1, b1, v2, g2, b2, wd, bd):
    raise NotImplementedError("write your pallas kernel here")


# speedup vs baseline: 1.3630x; 1.3630x over previous
import functools

import jax
import jax.numpy as jnp
from jax.experimental import pallas as pl
from jax.experimental.pallas import tpu as pltpu

_K = 3
_DIL = 4
_PAD = _DIL * (_K - 1)


def _fused_block_kernel(x_ref, w1_ref, b1_ref, w2_ref, b2_ref, wd_ref, bd_ref,
                        o_ref, h_ref):
    lp = x_ref.shape[-1]
    l_out = lp - _PAD
    c_out = o_ref.shape[1]

    xb = x_ref[0].astype(jnp.bfloat16)

    acc = jnp.dot(w1_ref[0], xb[:, 0:l_out],
                  preferred_element_type=jnp.float32)
    acc += jnp.dot(w1_ref[1], xb[:, _DIL:_DIL + l_out],
                   preferred_element_type=jnp.float32)
    acc += jnp.dot(w1_ref[2], xb[:, 2 * _DIL:2 * _DIL + l_out],
                   preferred_element_type=jnp.float32)
    h = jnp.maximum(acc + b1_ref[...], 0.0).astype(jnp.bfloat16)

    h_ref[:, :_PAD] = jnp.zeros((c_out, _PAD), jnp.bfloat16)
    h_ref[:, _PAD:] = h
    hb = h_ref[...]

    acc2 = jnp.dot(w2_ref[0], hb[:, 0:l_out],
                   preferred_element_type=jnp.float32)
    acc2 += jnp.dot(w2_ref[1], hb[:, _DIL:_DIL + l_out],
                    preferred_element_type=jnp.float32)
    acc2 += jnp.dot(w2_ref[2], hb[:, 2 * _DIL:2 * _DIL + l_out],
                    preferred_element_type=jnp.float32)
    out = jnp.maximum(acc2 + b2_ref[...], 0.0)

    res = jnp.dot(wd_ref[...], xb[:, _PAD:],
                  preferred_element_type=jnp.float32) + bd_ref[...]
    o_ref[0] = jnp.maximum(out + res, 0.0)


def kernel(x, v1, g1, b1, v2, g2, b2, wd, bd):
    n, c_in, l = x.shape
    c_out = v1.shape[0]

    def wn(v, g):
        norm = jnp.sqrt(jnp.sum(v.astype(jnp.float32) ** 2, axis=(1, 2),
                                keepdims=True))
        return g[:, None, None] * v.astype(jnp.float32) / norm

    w1_t = jnp.transpose(wn(v1, g1), (2, 0, 1)).astype(jnp.bfloat16)
    w2_t = jnp.transpose(wn(v2, g2), (2, 0, 1)).astype(jnp.bfloat16)
    wd2 = wd.reshape(c_out, c_in).astype(jnp.bfloat16)
    b1c = b1.reshape(c_out, 1)
    b2c = b2.reshape(c_out, 1)
    bdc = bd.reshape(c_out, 1)

    x_pad = jnp.pad(x, ((0, 0), (0, 0), (_PAD, 0)))
    lp = l + _PAD

    return pl.pallas_call(
        _fused_block_kernel,
        out_shape=jax.ShapeDtypeStruct((n, c_out, l), x.dtype),
        grid_spec=pltpu.PrefetchScalarGridSpec(
            num_scalar_prefetch=0,
            grid=(n,),
            in_specs=[
                pl.BlockSpec((1, c_in, lp), lambda b_: (b_, 0, 0)),
                pl.BlockSpec((_K, c_out, c_in), lambda b_: (0, 0, 0)),
                pl.BlockSpec((c_out, 1), lambda b_: (0, 0)),
                pl.BlockSpec((_K, c_out, c_out), lambda b_: (0, 0, 0)),
                pl.BlockSpec((c_out, 1), lambda b_: (0, 0)),
                pl.BlockSpec((c_out, c_in), lambda b_: (0, 0)),
                pl.BlockSpec((c_out, 1), lambda b_: (0, 0)),
            ],
            out_specs=pl.BlockSpec((1, c_out, l), lambda b_: (b_, 0, 0)),
            scratch_shapes=[pltpu.VMEM((c_out, lp), jnp.bfloat16)],
        ),
        compiler_params=pltpu.CompilerParams(dimension_semantics=("parallel",)),
    )(x_pad, w1_t, b1c, w2_t, b2c, wd2, bdc)


# speedup vs baseline: 1.5200x; 1.1152x over previous
import functools

import jax
import jax.numpy as jnp
from jax.experimental import pallas as pl
from jax.experimental.pallas import tpu as pltpu

_K = 3
_DIL = 4
_PAD = _DIL * (_K - 1)


def _fused_block_kernel(x_ref, w1_ref, b1_ref, w2_ref, b2_ref, wd_ref, bd_ref,
                        o_ref, xs_ref, h_ref):
    l_out = o_ref.shape[-1]
    c_in = x_ref.shape[1]
    c_out = o_ref.shape[1]

    xs_ref[:, :_PAD] = jnp.zeros((c_in, _PAD), jnp.bfloat16)
    xs_ref[:, _PAD:] = x_ref[0].astype(jnp.bfloat16)
    xb = xs_ref[...]

    acc = jnp.dot(w1_ref[0], xb[:, 0:l_out],
                  preferred_element_type=jnp.float32)
    acc += jnp.dot(w1_ref[1], xb[:, _DIL:_DIL + l_out],
                   preferred_element_type=jnp.float32)
    acc += jnp.dot(w1_ref[2], xb[:, 2 * _DIL:2 * _DIL + l_out],
                   preferred_element_type=jnp.float32)
    h = jnp.maximum(acc + b1_ref[...], 0.0).astype(jnp.bfloat16)

    h_ref[:, :_PAD] = jnp.zeros((c_out, _PAD), jnp.bfloat16)
    h_ref[:, _PAD:] = h
    hb = h_ref[...]

    acc2 = jnp.dot(w2_ref[0], hb[:, 0:l_out],
                   preferred_element_type=jnp.float32)
    acc2 += jnp.dot(w2_ref[1], hb[:, _DIL:_DIL + l_out],
                    preferred_element_type=jnp.float32)
    acc2 += jnp.dot(w2_ref[2], hb[:, 2 * _DIL:2 * _DIL + l_out],
                    preferred_element_type=jnp.float32)
    out = jnp.maximum(acc2 + b2_ref[...], 0.0)

    res = jnp.dot(wd_ref[...], xb[:, _PAD:],
                  preferred_element_type=jnp.float32) + bd_ref[...]
    o_ref[0] = jnp.maximum(out + res, 0.0)


def kernel(x, v1, g1, b1, v2, g2, b2, wd, bd):
    n, c_in, l = x.shape
    c_out = v1.shape[0]

    def wn(v, g):
        norm = jnp.sqrt(jnp.sum(v.astype(jnp.float32) ** 2, axis=(1, 2),
                                keepdims=True))
        return g[:, None, None] * v.astype(jnp.float32) / norm

    w1_t = jnp.transpose(wn(v1, g1), (2, 0, 1)).astype(jnp.bfloat16)
    w2_t = jnp.transpose(wn(v2, g2), (2, 0, 1)).astype(jnp.bfloat16)
    wd2 = wd.reshape(c_out, c_in).astype(jnp.bfloat16)
    b1c = b1.reshape(c_out, 1)
    b2c = b2.reshape(c_out, 1)
    bdc = bd.reshape(c_out, 1)

    lp = l + _PAD

    return pl.pallas_call(
        _fused_block_kernel,
        out_shape=jax.ShapeDtypeStruct((n, c_out, l), x.dtype),
        grid_spec=pltpu.PrefetchScalarGridSpec(
            num_scalar_prefetch=0,
            grid=(n,),
            in_specs=[
                pl.BlockSpec((1, c_in, l), lambda b_: (b_, 0, 0)),
                pl.BlockSpec((_K, c_out, c_in), lambda b_: (0, 0, 0)),
                pl.BlockSpec((c_out, 1), lambda b_: (0, 0)),
                pl.BlockSpec((_K, c_out, c_out), lambda b_: (0, 0, 0)),
                pl.BlockSpec((c_out, 1), lambda b_: (0, 0)),
                pl.BlockSpec((c_out, c_in), lambda b_: (0, 0)),
                pl.BlockSpec((c_out, 1), lambda b_: (0, 0)),
            ],
            out_specs=pl.BlockSpec((1, c_out, l), lambda b_: (b_, 0, 0)),
            scratch_shapes=[pltpu.VMEM((c_in, lp), jnp.bfloat16),
                            pltpu.VMEM((c_out, lp), jnp.bfloat16)],
        ),
        compiler_params=pltpu.CompilerParams(dimension_semantics=("parallel",)),
    )(x, w1_t, b1c, w2_t, b2c, wd2, bdc)


# speedup vs baseline: 1.9375x; 1.2747x over previous
import jax
import jax.numpy as jnp
from jax.experimental import pallas as pl
from jax.experimental.pallas import tpu as pltpu

_K = 3
_DIL = 4
_PAD = _DIL * (_K - 1)


def _fused_block_kernel(x_ref, w1_ref, b1_ref, w2_ref, b2_ref, wd_ref, bd_ref,
                        o_ref, xc_ref, hc_ref):
    l_out = o_ref.shape[-1]
    c_in = x_ref.shape[1]
    c_out = o_ref.shape[1]

    xb = x_ref[0].astype(jnp.bfloat16)
    for j in range(_K):
        shift = (_K - 1 - j) * _DIL
        r0 = j * c_in
        if shift:
            xc_ref[r0:r0 + c_in, :shift] = jnp.zeros((c_in, shift),
                                                     jnp.bfloat16)
            xc_ref[r0:r0 + c_in, shift:] = xb[:, :l_out - shift]
        else:
            xc_ref[r0:r0 + c_in, :] = xb

    acc = jnp.dot(w1_ref[...], xc_ref[...],
                  preferred_element_type=jnp.float32)
    h = jnp.maximum(acc + b1_ref[...], 0.0).astype(jnp.bfloat16)

    for j in range(_K):
        shift = (_K - 1 - j) * _DIL
        r0 = j * c_out
        if shift:
            hc_ref[r0:r0 + c_out, :shift] = jnp.zeros((c_out, shift),
                                                      jnp.bfloat16)
            hc_ref[r0:r0 + c_out, shift:] = h[:, :l_out - shift]
        else:
            hc_ref[r0:r0 + c_out, :] = h

    acc2 = jnp.dot(w2_ref[...], hc_ref[...],
                   preferred_element_type=jnp.float32)
    out = jnp.maximum(acc2 + b2_ref[...], 0.0)

    res = jnp.dot(wd_ref[...], xc_ref[(_K - 1) * c_in:, :],
                  preferred_element_type=jnp.float32) + bd_ref[...]
    o_ref[0] = jnp.maximum(out + res, 0.0)


def kernel(x, v1, g1, b1, v2, g2, b2, wd, bd):
    n, c_in, l = x.shape
    c_out = v1.shape[0]

    def wn(v, g):
        norm = jnp.sqrt(jnp.sum(v.astype(jnp.float32) ** 2, axis=(1, 2),
                                keepdims=True))
        return g[:, None, None] * v.astype(jnp.float32) / norm

    w1_s = jnp.transpose(wn(v1, g1), (0, 2, 1)).reshape(
        c_out, _K * c_in).astype(jnp.bfloat16)
    w2_s = jnp.transpose(wn(v2, g2), (0, 2, 1)).reshape(
        c_out, _K * c_out).astype(jnp.bfloat16)
    wd2 = wd.reshape(c_out, c_in).astype(jnp.bfloat16)
    b1c = b1.reshape(c_out, 1)
    b2c = b2.reshape(c_out, 1)
    bdc = bd.reshape(c_out, 1)

    return pl.pallas_call(
        _fused_block_kernel,
        out_shape=jax.ShapeDtypeStruct((n, c_out, l), x.dtype),
        grid_spec=pltpu.PrefetchScalarGridSpec(
            num_scalar_prefetch=0,
            grid=(n,),
            in_specs=[
                pl.BlockSpec((1, c_in, l), lambda b_: (b_, 0, 0)),
                pl.BlockSpec((c_out, _K * c_in), lambda b_: (0, 0)),
                pl.BlockSpec((c_out, 1), lambda b_: (0, 0)),
                pl.BlockSpec((c_out, _K * c_out), lambda b_: (0, 0)),
                pl.BlockSpec((c_out, 1), lambda b_: (0, 0)),
                pl.BlockSpec((c_out, c_in), lambda b_: (0, 0)),
                pl.BlockSpec((c_out, 1), lambda b_: (0, 0)),
            ],
            out_specs=pl.BlockSpec((1, c_out, l), lambda b_: (b_, 0, 0)),
            scratch_shapes=[pltpu.VMEM((_K * c_in, l), jnp.bfloat16),
                            pltpu.VMEM((_K * c_out, l), jnp.bfloat16)],
        ),
        compiler_params=pltpu.CompilerParams(dimension_semantics=("parallel",)),
    )(x, w1_s, b1c, w2_s, b2c, wd2, bdc)
